# baseline (speedup 1.0000x reference)
import jax
import jax.numpy as jnp
from jax.experimental import pallas as pl

ALPHA = 1.702
LIMIT = 7.0
FC = 512


def _moe_body(x_ref, wg_ref, wu_ref, w2_ref, rw_ref, b1g_ref, b1u_ref,
              b2_ref, out_ref):
    e = pl.program_id(0)

    @pl.when(e == 0)
    def _init():
        out_ref[...] = jnp.zeros_like(out_ref)

    x = x_ref[...]
    f = w2_ref.shape[1]
    rw_col = rw_ref[0, 0, :].reshape(-1, 1)
    for c in range(f // FC):
        sl = pl.ds(c * FC, FC)
        g = jnp.dot(x, wg_ref[0, :, sl], preferred_element_type=jnp.float32)
        u = jnp.dot(x, wu_ref[0, :, sl], preferred_element_type=jnp.float32)
        g = g + b1g_ref[0, 0, sl][None, :]
        u = u + b1u_ref[0, 0, sl][None, :]
        g = jnp.minimum(g, LIMIT)
        u = jnp.clip(u, -LIMIT, LIMIT)
        glu = g * jax.nn.sigmoid(g * ALPHA)
        h = ((u + 1.0) * glu * rw_col).astype(jnp.bfloat16)
        out_ref[...] += jnp.dot(
            h, w2_ref[0, sl, :], preferred_element_type=jnp.float32
        )
    out_ref[...] += rw_col * b2_ref[0, 0, :][None, :]


@jax.jit
def kernel(hidden_states, router_indices, routing_weights, gate_up_proj,
           gate_up_proj_bias, down_proj, down_proj_bias):
    bsz, tt, hid = hidden_states.shape
    num_e, _, f2 = gate_up_proj.shape
    f = f2 // 2
    tok = bsz * tt

    x = hidden_states.reshape(tok, hid).astype(jnp.bfloat16)
    w1b = gate_up_proj.astype(jnp.bfloat16)
    wg = w1b[:, :, 0::2]
    wu = w1b[:, :, 1::2]
    w2 = down_proj.astype(jnp.bfloat16)
    b1g = gate_up_proj_bias[:, 0::2].reshape(num_e, 1, f)
    b1u = gate_up_proj_bias[:, 1::2].reshape(num_e, 1, f)
    b2 = down_proj_bias.reshape(num_e, 1, hid)
    rw = routing_weights.T.reshape(num_e, 1, tok)

    out = pl.pallas_call(
        _moe_body,
        grid=(num_e,),
        in_specs=[
            pl.BlockSpec((tok, hid), lambda e: (0, 0)),
            pl.BlockSpec((1, hid, f), lambda e: (e, 0, 0)),
            pl.BlockSpec((1, hid, f), lambda e: (e, 0, 0)),
            pl.BlockSpec((1, f, hid), lambda e: (e, 0, 0)),
            pl.BlockSpec((1, 1, tok), lambda e: (e, 0, 0)),
            pl.BlockSpec((1, 1, f), lambda e: (e, 0, 0)),
            pl.BlockSpec((1, 1, f), lambda e: (e, 0, 0)),
            pl.BlockSpec((1, 1, hid), lambda e: (e, 0, 0)),
        ],
        out_specs=pl.BlockSpec((tok, hid), lambda e: (0, 0)),
        out_shape=jax.ShapeDtypeStruct((tok, hid), jnp.float32),
    )(x, wg, wu, w2, rw, b1g, b1u, b2)
    return out.reshape(bsz, tt, hid)


# speedup vs baseline: 4.0584x; 4.0584x over previous
import jax
import jax.numpy as jnp
from jax.experimental import pallas as pl

ALPHA = 1.702
LIMIT = 7.0
FC = 512


def _moe_body(x_ref, w1_ref, w2_ref, rw_ref, b1_ref, b2_ref, out_ref):
    e = pl.program_id(0)

    @pl.when(e == 0)
    def _init():
        out_ref[...] = jnp.zeros_like(out_ref)

    x = x_ref[...]
    f = w2_ref.shape[1]
    rw_col = rw_ref[0, 0, :].reshape(-1, 1)
    for c in range(f // FC):
        sl = pl.ds(c * FC, FC)
        su = pl.ds(f + c * FC, FC)
        g = jnp.dot(x, w1_ref[0, :, sl], preferred_element_type=jnp.float32)
        u = jnp.dot(x, w1_ref[0, :, su], preferred_element_type=jnp.float32)
        g = g + b1_ref[0, 0, sl][None, :]
        u = u + b1_ref[0, 0, su][None, :]
        g = jnp.minimum(g, LIMIT)
        u = jnp.clip(u, -LIMIT, LIMIT)
        glu = g * jax.nn.sigmoid(g * ALPHA)
        h = ((u + 1.0) * glu * rw_col).astype(jnp.bfloat16)
        out_ref[...] += jnp.dot(
            h, w2_ref[0, sl, :], preferred_element_type=jnp.float32
        )
    out_ref[...] += rw_col * b2_ref[0, 0, :][None, :]


@jax.jit
def kernel(hidden_states, router_indices, routing_weights, gate_up_proj,
           gate_up_proj_bias, down_proj, down_proj_bias):
    bsz, tt, hid = hidden_states.shape
    num_e, _, f2 = gate_up_proj.shape
    f = f2 // 2
    tok = bsz * tt

    x = hidden_states.reshape(tok, hid).astype(jnp.bfloat16)
    idx = jnp.concatenate([jnp.arange(0, f2, 2), jnp.arange(1, f2, 2)])
    w1 = jnp.take(gate_up_proj, idx, axis=2).astype(jnp.bfloat16)
    w2 = down_proj.astype(jnp.bfloat16)
    b1i = gate_up_proj_bias.reshape(num_e, f, 2)
    b1 = jnp.concatenate([b1i[..., 0], b1i[..., 1]], axis=-1).reshape(num_e, 1, f2)
    b2 = down_proj_bias.reshape(num_e, 1, hid)
    rw = routing_weights.T.reshape(num_e, 1, tok)

    out = pl.pallas_call(
        _moe_body,
        grid=(num_e,),
        in_specs=[
            pl.BlockSpec((tok, hid), lambda e: (0, 0)),
            pl.BlockSpec((1, hid, f2), lambda e: (e, 0, 0)),
            pl.BlockSpec((1, f, hid), lambda e: (e, 0, 0)),
            pl.BlockSpec((1, 1, tok), lambda e: (e, 0, 0)),
            pl.BlockSpec((1, 1, f2), lambda e: (e, 0, 0)),
            pl.BlockSpec((1, 1, hid), lambda e: (e, 0, 0)),
        ],
        out_specs=pl.BlockSpec((tok, hid), lambda e: (0, 0)),
        out_shape=jax.ShapeDtypeStruct((tok, hid), jnp.float32),
    )(x, w1, w2, rw, b1, b2)
    return out.reshape(bsz, tt, hid)


# speedup vs baseline: 4.4916x; 1.1068x over previous
import jax
import jax.numpy as jnp
from jax.experimental import pallas as pl

ALPHA = 1.702
LIMIT = 7.0
FC = 512


def _moe_body(x_ref, w1_ref, w2_ref, rw_ref, b1_ref, b2_ref, out_ref):
    e = pl.program_id(0)

    @pl.when(e == 0)
    def _init():
        out_ref[...] = jnp.zeros_like(out_ref)

    x = x_ref[...]
    f = w2_ref.shape[1]
    rw_col = rw_ref[0, 0, :].reshape(-1, 1)
    for c in range(f // FC):
        sl = pl.ds(c * FC, FC)
        su = pl.ds(f + c * FC, FC)
        g = jnp.dot(x, w1_ref[0, :, sl], preferred_element_type=jnp.float32)
        u = jnp.dot(x, w1_ref[0, :, su], preferred_element_type=jnp.float32)
        g = g + b1_ref[0, 0, sl][None, :]
        u = u + b1_ref[0, 0, su][None, :]
        g = jnp.minimum(g, LIMIT)
        u = jnp.clip(u, -LIMIT, LIMIT)
        glu = g * jax.nn.sigmoid(g * ALPHA)
        h = ((u + 1.0) * glu * rw_col).astype(jnp.bfloat16)
        w2c = w2_ref[0, sl, :].astype(jnp.bfloat16)
        out_ref[...] += jnp.dot(h, w2c, preferred_element_type=jnp.float32)
    out_ref[...] += rw_col * b2_ref[0, 0, :][None, :]


@jax.jit
def kernel(hidden_states, router_indices, routing_weights, gate_up_proj,
           gate_up_proj_bias, down_proj, down_proj_bias):
    bsz, tt, hid = hidden_states.shape
    num_e, _, f2 = gate_up_proj.shape
    f = f2 // 2
    tok = bsz * tt

    x = hidden_states.reshape(tok, hid).astype(jnp.bfloat16)
    gup = gate_up_proj.reshape(num_e, hid, f, 2)
    w1 = jnp.concatenate([gup[..., 0], gup[..., 1]], axis=-1).astype(jnp.bfloat16)
    b1i = gate_up_proj_bias.reshape(num_e, f, 2)
    b1 = jnp.concatenate([b1i[..., 0], b1i[..., 1]], axis=-1).reshape(num_e, 1, f2)
    b2 = down_proj_bias.reshape(num_e, 1, hid)
    rw = routing_weights.T.reshape(num_e, 1, tok)

    out = pl.pallas_call(
        _moe_body,
        grid=(num_e,),
        in_specs=[
            pl.BlockSpec((tok, hid), lambda e: (0, 0)),
            pl.BlockSpec((1, hid, f2), lambda e: (e, 0, 0)),
            pl.BlockSpec((1, f, hid), lambda e: (e, 0, 0)),
            pl.BlockSpec((1, 1, tok), lambda e: (e, 0, 0)),
            pl.BlockSpec((1, 1, f2), lambda e: (e, 0, 0)),
            pl.BlockSpec((1, 1, hid), lambda e: (e, 0, 0)),
        ],
        out_specs=pl.BlockSpec((tok, hid), lambda e: (0, 0)),
        out_shape=jax.ShapeDtypeStruct((tok, hid), jnp.float32),
    )(x, w1, down_proj, rw, b1, b2)
    return out.reshape(bsz, tt, hid)


# speedup vs baseline: 4.5510x; 1.0132x over previous
import jax
import jax.numpy as jnp
from jax.experimental import pallas as pl
from jax.experimental.pallas import tpu as pltpu

ALPHA = 1.702
LIMIT = 7.0
FC = 512


def _moe_body(x_ref, w1_ref, w2_ref, rw_ref, b1_ref, b2_ref, out_ref, xb_ref):
    e = pl.program_id(0)

    @pl.when(e == 0)
    def _init():
        out_ref[...] = jnp.zeros_like(out_ref)
        xb_ref[...] = x_ref[...].astype(jnp.bfloat16)

    x = xb_ref[...]
    f = w2_ref.shape[1]
    rw_col = rw_ref[0, 0, :].reshape(-1, 1)
    for c in range(f // FC):
        sl = pl.ds(c * FC, FC)
        su = pl.ds(f + c * FC, FC)
        g = jnp.dot(x, w1_ref[0, :, sl], preferred_element_type=jnp.float32)
        u = jnp.dot(x, w1_ref[0, :, su], preferred_element_type=jnp.float32)
        g = g + b1_ref[0, 0, sl][None, :]
        u = u + b1_ref[0, 0, su][None, :]
        g = jnp.minimum(g, LIMIT)
        u = jnp.clip(u, -LIMIT, LIMIT)
        glu = g * jax.nn.sigmoid(g * ALPHA)
        h = ((u + 1.0) * glu * rw_col).astype(jnp.bfloat16)
        w2c = w2_ref[0, sl, :].astype(jnp.bfloat16)
        out_ref[...] += jnp.dot(h, w2c, preferred_element_type=jnp.float32)
    out_ref[...] += rw_col * b2_ref[0, 0, :][None, :]


@jax.jit
def kernel(hidden_states, router_indices, routing_weights, gate_up_proj,
           gate_up_proj_bias, down_proj, down_proj_bias):
    bsz, tt, hid = hidden_states.shape
    num_e, _, f2 = gate_up_proj.shape
    f = f2 // 2
    tok = bsz * tt

    x = hidden_states.reshape(tok, hid)
    gup = gate_up_proj.reshape(num_e, hid, f, 2)
    w1 = jnp.concatenate([gup[..., 0], gup[..., 1]], axis=-1).astype(jnp.bfloat16)
    b1i = gate_up_proj_bias.reshape(num_e, f, 2)
    b1 = jnp.concatenate([b1i[..., 0], b1i[..., 1]], axis=-1).reshape(num_e, 1, f2)
    b2 = down_proj_bias.reshape(num_e, 1, hid)
    rw = routing_weights.T.reshape(num_e, 1, tok)

    out = pl.pallas_call(
        _moe_body,
        grid=(num_e,),
        in_specs=[
            pl.BlockSpec((tok, hid), lambda e: (0, 0)),
            pl.BlockSpec((1, hid, f2), lambda e: (e, 0, 0)),
            pl.BlockSpec((1, f, hid), lambda e: (e, 0, 0)),
            pl.BlockSpec((1, 1, tok), lambda e: (e, 0, 0)),
            pl.BlockSpec((1, 1, f2), lambda e: (e, 0, 0)),
            pl.BlockSpec((1, 1, hid), lambda e: (e, 0, 0)),
        ],
        out_specs=pl.BlockSpec((tok, hid), lambda e: (0, 0)),
        out_shape=jax.ShapeDtypeStruct((tok, hid), jnp.float32),
        scratch_shapes=[pltpu.VMEM((tok, hid), jnp.bfloat16)],
    )(x, w1, down_proj, rw, b1, b2)
    return out.reshape(bsz, tt, hid)
